# Initial kernel scaffold; baseline (speedup 1.0000x reference)
import jax
import jax.numpy as jnp
from jax.experimental import pallas as pl


def kernel(idx, bigram_log_probs, trigram_log_probs, fourgram_log_probs):
    raise NotImplementedError("write your pallas kernel here")


# speedup vs baseline: 1.0406x; 1.0406x over previous
import functools
import math

import jax
import jax.numpy as jnp
from jax import lax
from jax.experimental import pallas as pl
from jax.experimental.pallas import tpu as pltpu
from jax.experimental.pallas import tpu_sc as plsc

V = 65
B = 64
T = 512
C = 128
L = 16
LOG_W = math.log(1.0 / 3.0 + 1e-10)
LN2 = 0.6931471805599453
SQRT2 = 1.4142135623730951


def _softlog(x):
    bits = lax.bitcast_convert_type(x, jnp.int32)
    e = ((bits >> 23) & 0xFF) - 127
    m = lax.bitcast_convert_type((bits & 0x7FFFFF) | (127 << 23), jnp.float32)
    big = m > SQRT2
    m = jnp.where(big, m * 0.5, m)
    e = jnp.where(big, e + 1, e)
    s = (m - 1.0) / (m + 1.0)
    z = s * s
    p = s * (2.0 + z * (0.6666666666666666 + z * (0.4 + z * 0.2857142857142857)))
    return e.astype(jnp.float32) * LN2 + p


def _combine(a, b, c):
    return _softlog(jnp.exp(a) + jnp.exp(b) + jnp.exp(c)) + LOG_W


def _sc_body(idx_ref, bi_ref, tri_ref, four_ref, out_ref,
             idx_v, rbi_v, rtri_v, rfour_v,
             bi_rows, tri_rows, four_rows, out_v, sem_g):
    cid = lax.axis_index("c")
    sid = lax.axis_index("s")
    wid = sid * 2 + cid
    rows_per_tile = B // 32

    for rr in range(rows_per_tile):
        b = wid * rows_per_tile + rr
        pltpu.sync_copy(idx_ref.at[b], idx_v)

        def chunk_body(ci, _):
            tok0 = ci * C
            for g in range(C // L):
                tvec = lax.iota(jnp.int32, L) + (tok0 + g * L)
                i1 = plsc.load_gather(idx_v, [jnp.maximum(tvec - 1, 0)])
                i2 = plsc.load_gather(idx_v, [jnp.maximum(tvec - 2, 0)])
                i3 = plsc.load_gather(idx_v, [jnp.maximum(tvec - 3, 0)])
                rtri = i2 * V + i1
                rfour = i3 * (V * V) + rtri
                rbi_v[pl.ds(g * L, L)] = i1
                rtri_v[pl.ds(g * L, L)] = rtri
                rfour_v[pl.ds(g * L, L)] = rfour
            cp1 = pltpu.async_copy(bi_ref.at[rbi_v], bi_rows, sem_g)
            cp2 = pltpu.async_copy(tri_ref.at[rtri_v], tri_rows, sem_g)
            cp3 = pltpu.async_copy(four_ref.at[rfour_v], four_rows, sem_g)
            cp1.wait()
            cp2.wait()
            cp3.wait()
            @pl.when(ci == 0)
            def _zero_fill():
                zero = jnp.zeros((L,), jnp.float32)
                for buf, nrows in ((bi_rows, 1), (tri_rows, 2), (four_rows, 3)):
                    for r in range(nrows):
                        for j in range(V // L):
                            buf[r, pl.ds(j * L, L)] = zero
                        buf[r, pl.ds(V - L, L)] = zero
            def tok_body(t, _):
                for j in range(V // L):
                    sl = pl.ds(j * L, L)
                    out_v[t, sl] = _combine(bi_rows[t, sl], tri_rows[t, sl],
                                            four_rows[t, sl])
                return 0

            lax.fori_loop(0, C, tok_body, 0)
            colv = jnp.full((L,), V - 1, jnp.int32)
            for g in range(C // L):
                rows = lax.iota(jnp.int32, L) + g * L
                val = _combine(plsc.load_gather(bi_rows, [rows, colv]),
                               plsc.load_gather(tri_rows, [rows, colv]),
                               plsc.load_gather(four_rows, [rows, colv]))
                plsc.store_scatter(out_v, [rows, colv], val)
            pltpu.sync_copy(out_v, out_ref.at[pl.ds(b * T + tok0, C)])
            return 0

        lax.fori_loop(0, T // C, chunk_body, 0)


@functools.partial(
    pl.kernel,
    mesh=plsc.VectorSubcoreMesh(core_axis_name="c", subcore_axis_name="s"),
    out_type=jax.ShapeDtypeStruct((B * T, V), jnp.float32),
    compiler_params=pltpu.CompilerParams(
        needs_layout_passes=False, use_tc_tiling_on_sc=False),
    scratch_types=[
        pltpu.VMEM((T,), jnp.int32),
        pltpu.VMEM((C,), jnp.int32),
        pltpu.VMEM((C,), jnp.int32),
        pltpu.VMEM((C,), jnp.int32),
        pltpu.VMEM((C, V), jnp.float32),
        pltpu.VMEM((C, V), jnp.float32),
        pltpu.VMEM((C, V), jnp.float32),
        pltpu.VMEM((C, V), jnp.float32),
        pltpu.SemaphoreType.DMA,
    ],
)
def _ngram_sc_kernel(*refs):
    _sc_body(*refs)


def kernel(idx, bigram_log_probs, trigram_log_probs, fourgram_log_probs):
    idx32 = idx.astype(jnp.int32)
    tri2 = trigram_log_probs.reshape(V * V, V)
    four2 = fourgram_log_probs.reshape(V * V * V, V)
    out = _ngram_sc_kernel(idx32, bigram_log_probs, tri2, four2)
    return out.reshape(B, T, V)


# speedup vs baseline: 2.4446x; 2.3493x over previous
import functools
import math

import jax
import jax.numpy as jnp
from jax import lax
from jax.experimental import pallas as pl
from jax.experimental.pallas import tpu as pltpu
from jax.experimental.pallas import tpu_sc as plsc

V = 65
VP = 128
RP = 72
B = 64
T = 512
C = 128
L = 16
NW = 32
LOG_W = math.log(1.0 / 3.0 + 1e-10)
LN2 = 0.6931471805599453
SQRT2 = 1.4142135623730951


def _softlog(x):
    bits = lax.bitcast_convert_type(x, jnp.int32)
    e = ((bits >> 23) & 0xFF) - 127
    m = lax.bitcast_convert_type((bits & 0x7FFFFF) | (127 << 23), jnp.float32)
    big = m > SQRT2
    m = jnp.where(big, m * 0.5, m)
    e = jnp.where(big, e + 1, e)
    s = (m - 1.0) / (m + 1.0)
    z = s * s
    p = s * (2.0 + z * (0.6666666666666666 + z * (0.4 + z * 0.2857142857142857)))
    return e.astype(jnp.float32) * LN2 + p


def _combine(a, b, c):
    return _softlog(jnp.exp(a) + jnp.exp(b) + jnp.exp(c)) + LOG_W


def _sc_body(idx_ref, bi_ref, tri_ref, four_ref, out_ref,
             idx_v, rbi_v, rtri_v, rfour_v,
             bi_rows, tri_rows, four_rows, out_v, sem_g):
    cid = lax.axis_index("c")
    sid = lax.axis_index("s")
    wid = sid * 2 + cid
    rows_per_tile = B // NW

    for rr in range(rows_per_tile):
        b = wid * rows_per_tile + rr
        pltpu.sync_copy(idx_ref.at[b], idx_v)

        def chunk_body(ci, _):
            tok0 = ci * C
            for g in range(C // L):
                tvec = lax.iota(jnp.int32, L) + (tok0 + g * L)
                i1 = plsc.load_gather(idx_v, [jnp.maximum(tvec - 1, 0)])
                i2 = plsc.load_gather(idx_v, [jnp.maximum(tvec - 2, 0)])
                i3 = plsc.load_gather(idx_v, [jnp.maximum(tvec - 3, 0)])
                rbi_v[pl.ds(g * L, L)] = i1
                rtri_v[pl.ds(g * L, L)] = i2 * RP + i1
                rfour_v[pl.ds(g * L, L)] = (i3 * V + i2) * RP + i1
            cp1 = pltpu.async_copy(bi_ref.at[rbi_v], bi_rows, sem_g)
            cp2 = pltpu.async_copy(tri_ref.at[rtri_v], tri_rows, sem_g)
            cp3 = pltpu.async_copy(four_ref.at[rfour_v], four_rows, sem_g)
            cp1.wait()
            cp2.wait()
            cp3.wait()
            @pl.when(ci == 0)
            def _zero_fill():
                zero = jnp.zeros((L,), jnp.float32)
                for buf, nrows in ((bi_rows, 1), (tri_rows, 2), (four_rows, 3)):
                    for r in range(nrows):
                        for j in range(5):
                            buf[r, pl.ds(j * L, L)] = zero
            def tok_body(t, _):
                for j in range(5):
                    sl = pl.ds(j * L, L)
                    out_v[t, sl] = _combine(bi_rows[t, sl], tri_rows[t, sl],
                                            four_rows[t, sl])
                return 0

            lax.fori_loop(0, C, tok_body, 0)
            pltpu.sync_copy(out_v, out_ref.at[pl.ds(b * T + tok0, C)])
            return 0

        lax.fori_loop(0, T // C, chunk_body, 0)


@functools.partial(
    pl.kernel,
    mesh=plsc.VectorSubcoreMesh(core_axis_name="c", subcore_axis_name="s"),
    out_type=jax.ShapeDtypeStruct((B * T, VP), jnp.float32),
    compiler_params=pltpu.CompilerParams(needs_layout_passes=False),
    scratch_types=[
        pltpu.VMEM((T,), jnp.int32),
        pltpu.VMEM((C,), jnp.int32),
        pltpu.VMEM((C,), jnp.int32),
        pltpu.VMEM((C,), jnp.int32),
        pltpu.VMEM((C, VP), jnp.float32),
        pltpu.VMEM((C, VP), jnp.float32),
        pltpu.VMEM((C, VP), jnp.float32),
        pltpu.VMEM((C, VP), jnp.float32),
        pltpu.SemaphoreType.DMA,
    ],
)
def _ngram_sc_kernel(*refs):
    _sc_body(*refs)


def kernel(idx, bigram_log_probs, trigram_log_probs, fourgram_log_probs):
    idx32 = idx.astype(jnp.int32)
    bi_p = jnp.pad(bigram_log_probs, ((0, 0), (0, VP - V)))
    tri_p = jnp.pad(trigram_log_probs,
                    ((0, 0), (0, RP - V), (0, VP - V))).reshape(V * RP, VP)
    four_p = jnp.pad(fourgram_log_probs,
                     ((0, 0), (0, 0), (0, RP - V), (0, VP - V))
                     ).reshape(V * V * RP, VP)
    out = _ngram_sc_kernel(idx32, bi_p, tri_p, four_p)
    return out[:, :V].reshape(B, T, V)


# speedup vs baseline: 2.6593x; 1.0878x over previous
import functools
import math

import jax
import jax.numpy as jnp
from jax import lax
from jax.experimental import pallas as pl
from jax.experimental.pallas import tpu as pltpu
from jax.experimental.pallas import tpu_sc as plsc

V = 65
VP = 128
RP = 72
B = 64
T = 512
C = 64
NCH = T // C
L = 16
NW = 32
LOG_W = math.log(1.0 / 3.0 + 1e-10)
LN2 = 0.6931471805599453
SQRT2 = 1.4142135623730951


def _softlog(x):
    bits = lax.bitcast_convert_type(x, jnp.int32)
    e = ((bits >> 23) & 0xFF) - 127
    m = lax.bitcast_convert_type((bits & 0x7FFFFF) | (127 << 23), jnp.float32)
    big = m > SQRT2
    m = jnp.where(big, m * 0.5, m)
    e = jnp.where(big, e + 1, e)
    s = (m - 1.0) / (m + 1.0)
    z = s * s
    p = s * (2.0 + z * (0.6666666666666666 + z * (0.4 + z * 0.2857142857142857)))
    return e.astype(jnp.float32) * LN2 + p


def _combine(a, b, c):
    return _softlog(jnp.exp(a) + jnp.exp(b) + jnp.exp(c)) + LOG_W


def _sc_body(idx_ref, bi_ref, tri_ref, four_ref, out_ref,
             idx_v, rbi_v, rtri_v, rfour_v,
             bi_rows, tri_rows, four_rows, out_v, sem_g, sem_o):
    cid = lax.axis_index("c")
    sid = lax.axis_index("s")
    wid = sid * 2 + cid
    rows_per_tile = B // NW

    def idx_calc(ci):
        tok0 = ci * C
        for g in range(C // L):
            tvec = lax.iota(jnp.int32, L) + (tok0 + g * L)
            i1 = plsc.load_gather(idx_v, [jnp.maximum(tvec - 1, 0)])
            i2 = plsc.load_gather(idx_v, [jnp.maximum(tvec - 2, 0)])
            i3 = plsc.load_gather(idx_v, [jnp.maximum(tvec - 3, 0)])
            p = ci % 2
            rbi_v[p, pl.ds(g * L, L)] = i1
            rtri_v[p, pl.ds(g * L, L)] = i2 * RP + i1
            rfour_v[p, pl.ds(g * L, L)] = (i3 * V + i2) * RP + i1

    def issue_gathers(ci):
        p = ci % 2
        return (
            pltpu.async_copy(bi_ref.at[rbi_v.at[p]], bi_rows.at[p], sem_g.at[p]),
            pltpu.async_copy(tri_ref.at[rtri_v.at[p]], tri_rows.at[p], sem_g.at[p]),
            pltpu.async_copy(four_ref.at[rfour_v.at[p]], four_rows.at[p], sem_g.at[p]),
        )

    for rr in range(rows_per_tile):
        b = wid * rows_per_tile + rr
        pltpu.sync_copy(idx_ref.at[b], idx_v)

        idx_calc(0)
        cps = {0: issue_gathers(0)}
        out_cps = {}
        for ci in range(NCH):
            p = ci % 2
            if ci + 1 < NCH:
                idx_calc(ci + 1)
                cps[ci + 1] = issue_gathers(ci + 1)
            for cp in cps.pop(ci):
                cp.wait()
            if ci == 0:
                zero = jnp.zeros((L,), jnp.float32)
                for buf, nrows in ((bi_rows, 1), (tri_rows, 2), (four_rows, 3)):
                    for r in range(nrows):
                        for j in range(5):
                            buf[0, r, pl.ds(j * L, L)] = zero
            if ci >= 2:
                out_cps.pop(ci - 2).wait()

            def tok_body(t, _, p=p):
                for j in range(5):
                    sl = pl.ds(j * L, L)
                    out_v[p, t, sl] = _combine(bi_rows[p, t, sl],
                                               tri_rows[p, t, sl],
                                               four_rows[p, t, sl])
                return 0

            lax.fori_loop(0, C, tok_body, 0)
            out_cps[ci] = pltpu.async_copy(
                out_v.at[p], out_ref.at[pl.ds(b * T + ci * C, C)], sem_o.at[p])
        for ci in sorted(out_cps):
            out_cps.pop(ci).wait()


@functools.partial(
    pl.kernel,
    mesh=plsc.VectorSubcoreMesh(core_axis_name="c", subcore_axis_name="s"),
    out_type=jax.ShapeDtypeStruct((B * T, VP), jnp.float32),
    compiler_params=pltpu.CompilerParams(needs_layout_passes=False),
    scratch_types=[
        pltpu.VMEM((T,), jnp.int32),
        pltpu.VMEM((2, C), jnp.int32),
        pltpu.VMEM((2, C), jnp.int32),
        pltpu.VMEM((2, C), jnp.int32),
        pltpu.VMEM((2, C, VP), jnp.float32),
        pltpu.VMEM((2, C, VP), jnp.float32),
        pltpu.VMEM((2, C, VP), jnp.float32),
        pltpu.VMEM((2, C, VP), jnp.float32),
        pltpu.SemaphoreType.DMA((2,)),
        pltpu.SemaphoreType.DMA((2,)),
    ],
)
def _ngram_sc_kernel(*refs):
    _sc_body(*refs)


def kernel(idx, bigram_log_probs, trigram_log_probs, fourgram_log_probs):
    idx32 = idx.astype(jnp.int32)
    bi_p = jnp.pad(bigram_log_probs, ((0, 0), (0, VP - V)))
    tri_p = jnp.pad(trigram_log_probs,
                    ((0, 0), (0, RP - V), (0, VP - V))).reshape(V * RP, VP)
    four_p = jnp.pad(fourgram_log_probs,
                     ((0, 0), (0, 0), (0, RP - V), (0, VP - V))
                     ).reshape(V * V * RP, VP)
    out = _ngram_sc_kernel(idx32, bi_p, tri_p, four_p)
    return out[:, :V].reshape(B, T, V)


# speedup vs baseline: 2.9253x; 1.1000x over previous
import functools
import math

import jax
import jax.numpy as jnp
from jax import lax
from jax.experimental import pallas as pl
from jax.experimental.pallas import tpu as pltpu
from jax.experimental.pallas import tpu_sc as plsc

V = 65
VP = 128
RP = 72
B = 64
T = 512
C = 128
NCH = T // C
L = 16
NW = 32
LOG_W = math.log(1.0 / 3.0 + 1e-10)
LN2 = 0.6931471805599453
SQRT2 = 1.4142135623730951
SENT_TB1 = 65 * RP
SENT_TB0 = 65 * RP + 65
SENT_FOUR = 65


def _softlog(x):
    bits = lax.bitcast_convert_type(x, jnp.int32)
    e = ((bits >> 23) & 0xFF) - 127
    m = lax.bitcast_convert_type((bits & 0x7FFFFF) | (127 << 23), jnp.float32)
    big = m > SQRT2
    m = jnp.where(big, m * 0.5, m)
    e = jnp.where(big, e + 1, e)
    s = (m - 1.0) / (m + 1.0)
    z = s * s
    p = s * (2.0 + z * (0.6666666666666666 + z * (0.4 + z * 0.2857142857142857)))
    return e.astype(jnp.float32) * LN2 + p


def _sc_body(idx_ref, tb_ref, four_ref, out_ref,
             idx_v, rtb_v, rfour_v,
             tb_rows, four_rows, out_v, sem_g, sem_o):
    cid = lax.axis_index("c")
    sid = lax.axis_index("s")
    wid = sid * 2 + cid
    rows_per_tile = B // NW

    def idx_calc(ci):
        p = ci % 2
        tok0 = ci * C
        for g in range(C // L):
            tvec = lax.iota(jnp.int32, L) + (tok0 + g * L)
            i1 = plsc.load_gather(idx_v, [jnp.maximum(tvec - 1, 0)])
            i2 = plsc.load_gather(idx_v, [jnp.maximum(tvec - 2, 0)])
            i3 = plsc.load_gather(idx_v, [jnp.maximum(tvec - 3, 0)])
            rtb = i2 * RP + i1
            rfour = (i3 * V + i2) * RP + i1
            if ci == 0 and g == 0:
                rtb = jnp.where(tvec >= 2, rtb,
                                jnp.where(tvec == 1, SENT_TB1 + i1, SENT_TB0))
                rfour = jnp.where(tvec >= 3, rfour, SENT_FOUR)
            rtb_v[p, pl.ds(g * L, L)] = rtb
            rfour_v[p, pl.ds(g * L, L)] = rfour

    def issue_gathers(ci):
        p = ci % 2
        return (
            pltpu.async_copy(tb_ref.at[rtb_v.at[p]], tb_rows.at[p], sem_g.at[p]),
            pltpu.async_copy(four_ref.at[rfour_v.at[p]], four_rows.at[p], sem_g.at[p]),
        )

    for rr in range(rows_per_tile):
        b = wid * rows_per_tile + rr
        pltpu.sync_copy(idx_ref.at[b], idx_v)

        idx_calc(0)
        cps = {0: issue_gathers(0)}
        out_cps = {}
        for ci in range(NCH):
            p = ci % 2
            if ci + 1 < NCH:
                idx_calc(ci + 1)
                cps[ci + 1] = issue_gathers(ci + 1)
            for cp in cps.pop(ci):
                cp.wait()
            if ci >= 2:
                out_cps.pop(ci - 2).wait()

            def tok_body(t, _, p=p):
                for j in range(5):
                    sl = pl.ds(j * L, L)
                    s = tb_rows[p, t, sl] + jnp.exp(four_rows[p, t, sl])
                    out_v[p, t, sl] = _softlog(s) + LOG_W
                return 0

            lax.fori_loop(0, C, tok_body, 0)
            out_cps[ci] = pltpu.async_copy(
                out_v.at[p], out_ref.at[pl.ds(b * T + ci * C, C)], sem_o.at[p])
        for ci in sorted(out_cps):
            out_cps.pop(ci).wait()


@functools.partial(
    pl.kernel,
    mesh=plsc.VectorSubcoreMesh(core_axis_name="c", subcore_axis_name="s"),
    out_type=jax.ShapeDtypeStruct((B * T, VP), jnp.float32),
    compiler_params=pltpu.CompilerParams(needs_layout_passes=False),
    scratch_types=[
        pltpu.VMEM((T,), jnp.int32),
        pltpu.VMEM((2, C), jnp.int32),
        pltpu.VMEM((2, C), jnp.int32),
        pltpu.VMEM((2, C, VP), jnp.float32),
        pltpu.VMEM((2, C, VP), jnp.float32),
        pltpu.VMEM((2, C, VP), jnp.float32),
        pltpu.SemaphoreType.DMA((2,)),
        pltpu.SemaphoreType.DMA((2,)),
    ],
)
def _ngram_sc_kernel(*refs):
    _sc_body(*refs)


def kernel(idx, bigram_log_probs, trigram_log_probs, fourgram_log_probs):
    idx32 = idx.astype(jnp.int32)
    eb = jnp.exp(bigram_log_probs)
    tb = jnp.exp(trigram_log_probs) + eb[None, :, :]
    tb = jnp.concatenate([tb, (eb + 1.0)[None, :, :]], axis=0)
    tb_p = jnp.pad(tb, ((0, 0), (0, RP - V), (0, VP - V)))
    tb_p = tb_p.at[V, V, :].set(2.0)
    tb_p = tb_p.reshape((V + 1) * RP, VP)
    four_p = jnp.pad(fourgram_log_probs,
                     ((0, 0), (0, 0), (0, RP - V), (0, VP - V))
                     ).reshape(V * V * RP, VP)
    out = _ngram_sc_kernel(idx32, tb_p, four_p)
    return out[:, :V].reshape(B, T, V)
